# baseline (speedup 1.0000x reference)
import functools

import jax
import jax.numpy as jnp
from jax import lax
from jax.experimental import pallas as pl
from jax.experimental.pallas import tpu as pltpu
from jax.experimental.pallas import tpu_sc as plsc

_CHUNK = 16
_NBUF = 4


def _make_gather(n_total, d, n_workers, num_cores):
    n_per_w = n_total // n_workers
    n_chunks = n_per_w // _CHUNK
    n_groups = n_chunks // _NBUF

    mesh = plsc.VectorSubcoreMesh(core_axis_name="c", subcore_axis_name="s")

    @functools.partial(
        pl.kernel,
        mesh=mesh,
        out_type=jax.ShapeDtypeStruct((n_total, d), jnp.float32),
        scratch_types=[
            pltpu.VMEM((n_per_w,), jnp.int32),
            pltpu.VMEM((_NBUF, _CHUNK, d), jnp.float32),
        ] + [pltpu.SemaphoreType.DMA] * (2 * _NBUF),
    )
    def gather_kernel(table_hbm, idx_hbm, out_hbm, idx_v, rows_v, *sems):
        wid = lax.axis_index("s") * num_cores + lax.axis_index("c")
        base = wid * n_per_w
        pltpu.sync_copy(idx_hbm.at[pl.ds(base, n_per_w)], idx_v)

        gsem = sems[:_NBUF]
        wsem = sems[_NBUF:]

        def gather_desc(j, b):
            return pltpu.make_async_copy(
                table_hbm.at[idx_v.at[pl.ds(j * _CHUNK, _CHUNK)]],
                rows_v.at[b],
                gsem[b],
            )

        def write_desc(j, b):
            return pltpu.make_async_copy(
                rows_v.at[b],
                out_hbm.at[pl.ds(base + j * _CHUNK, _CHUNK)],
                wsem[b],
            )

        for b in range(_NBUF):
            gather_desc(b, b).start()

        def body(t, carry):
            j = t * _NBUF
            for b in range(_NBUF):
                gather_desc(j + b, b).wait()
                write_desc(j + b, b).start()
            for b in range(_NBUF):
                write_desc(j + b, b).wait()
                gather_desc(j + _NBUF + b, b).start()
            return carry

        lax.fori_loop(0, n_groups - 1, body, 0)

        j = (n_groups - 1) * _NBUF
        for b in range(_NBUF):
            gather_desc(j + b, b).wait()
            write_desc(j + b, b).start()
        for b in range(_NBUF):
            write_desc(j + b, b).wait()

    return gather_kernel


def kernel(position_embedding, positional_index_batch):
    b, s = positional_index_batch.shape
    p, d = position_embedding.shape
    n_total = b * s
    idx_flat = positional_index_batch.reshape(n_total).astype(jnp.int32)

    info = plsc.get_sparse_core_info()
    n_workers = info.num_cores * info.num_subcores

    out_flat = _make_gather(n_total, d, n_workers, info.num_cores)(
        position_embedding, idx_flat
    )
    return out_flat.reshape(b, s, d)
